# baseline (speedup 1.0000x reference)
import functools

import jax
import jax.numpy as jnp
from jax import lax
from jax.experimental import pallas as pl
from jax.experimental.pallas import tpu as pltpu
from jax.experimental.pallas import tpu_sc as plsc

VOCAB = 100000
EMBED_DIM = 300
EMBED_PAD = 304
BATCH = 4096
SEQ = 50

N_IDX = BATCH * SEQ
NUM_WORKERS = 32
PER_WORKER = N_IDX // NUM_WORKERS
CHUNK = 128
N_CHUNKS = PER_WORKER // CHUNK

_mesh = plsc.VectorSubcoreMesh(core_axis_name="c", subcore_axis_name="s")


@functools.partial(
    pl.kernel,
    mesh=_mesh,
    out_type=jax.ShapeDtypeStruct((N_IDX, EMBED_PAD), jnp.float32),
    scratch_types=[
        pltpu.VMEM((CHUNK,), jnp.int32),
        pltpu.VMEM((CHUNK, EMBED_PAD), jnp.float32),
        pltpu.SemaphoreType.DMA,
    ],
    compiler_params=pltpu.CompilerParams(use_tc_tiling_on_sc=False),
)
def _gather_sc(idx_hbm, table_hbm, out_hbm, idx_v, rows_v, sem):
    wid = lax.axis_index("s") * 2 + lax.axis_index("c")
    base = wid * PER_WORKER

    def chunk_body(j, carry):
        off = base + j * CHUNK
        pltpu.sync_copy(idx_hbm.at[pl.ds(off, CHUNK)], idx_v)
        pltpu.async_copy(table_hbm.at[idx_v], rows_v, sem).wait()
        pltpu.sync_copy(rows_v, out_hbm.at[pl.ds(off, CHUNK)])
        return carry

    lax.fori_loop(0, N_CHUNKS, chunk_body, 0)


def kernel(batch, table):
    flat = batch.reshape(N_IDX)
    tpad = jnp.pad(table, ((0, 0), (0, EMBED_PAD - EMBED_DIM)))
    outp = _gather_sc(flat, tpad)
    return outp[:, :EMBED_DIM].reshape(BATCH, SEQ, EMBED_DIM)


# speedup vs baseline: 1.6001x; 1.6001x over previous
import functools

import jax
import jax.numpy as jnp
from jax import lax
from jax.experimental import pallas as pl
from jax.experimental.pallas import tpu as pltpu
from jax.experimental.pallas import tpu_sc as plsc

VOCAB = 100000
EMBED_DIM = 300
EMBED_PAD = 384
BATCH = 4096
SEQ = 50
SEQ_PAD = 56

N_IDX = BATCH * SEQ_PAD
NUM_WORKERS = 32
PER_WORKER = N_IDX // NUM_WORKERS
CHUNK = 128
N_CHUNKS = PER_WORKER // CHUNK

_mesh = plsc.VectorSubcoreMesh(core_axis_name="c", subcore_axis_name="s")


@functools.partial(
    pl.kernel,
    mesh=_mesh,
    out_type=jax.ShapeDtypeStruct((N_IDX, EMBED_PAD), jnp.float32),
    scratch_types=[
        pltpu.VMEM((CHUNK,), jnp.int32),
        pltpu.VMEM((CHUNK, EMBED_PAD), jnp.float32),
        pltpu.SemaphoreType.DMA,
    ],
)
def _gather_sc(idx_hbm, table_hbm, out_hbm, idx_v, rows_v, sem):
    wid = lax.axis_index("s") * 2 + lax.axis_index("c")
    base = wid * PER_WORKER

    def chunk_body(j, carry):
        off = base + j * CHUNK
        pltpu.sync_copy(idx_hbm.at[pl.ds(off, CHUNK)], idx_v)
        pltpu.async_copy(table_hbm.at[idx_v], rows_v, sem).wait()
        pltpu.sync_copy(rows_v, out_hbm.at[pl.ds(off, CHUNK)])
        return carry

    lax.fori_loop(0, N_CHUNKS, chunk_body, 0)


_PAD_ROWS = 2000


def _pad_body(t_ref, o_ref):
    o_ref[:, :EMBED_DIM] = t_ref[...]
    o_ref[:, EMBED_DIM:] = jnp.zeros((_PAD_ROWS, EMBED_PAD - EMBED_DIM),
                                     jnp.float32)


_tc_pad = pl.pallas_call(
    _pad_body,
    grid=(VOCAB // _PAD_ROWS,),
    in_specs=[pl.BlockSpec((_PAD_ROWS, EMBED_DIM), lambda i: (i, 0))],
    out_specs=pl.BlockSpec((_PAD_ROWS, EMBED_PAD), lambda i: (i, 0)),
    out_shape=jax.ShapeDtypeStruct((VOCAB, EMBED_PAD), jnp.float32),
)

_SLC_B = 8


def _slice_body(p_ref, o_ref):
    o_ref[...] = p_ref[...].reshape(_SLC_B, SEQ_PAD, EMBED_PAD)


_tc_slice = pl.pallas_call(
    _slice_body,
    grid=(BATCH // _SLC_B,),
    in_specs=[pl.BlockSpec((_SLC_B * SEQ_PAD, EMBED_PAD), lambda i: (i, 0))],
    out_specs=pl.BlockSpec((_SLC_B, SEQ_PAD, EMBED_PAD), lambda i: (i, 0, 0)),
    out_shape=jax.ShapeDtypeStruct((BATCH, SEQ, EMBED_DIM), jnp.float32),
)


def kernel(batch, table):
    idxp = jnp.pad(batch, ((0, 0), (0, SEQ_PAD - SEQ)), mode="edge")
    flat = idxp.reshape(N_IDX)
    tpad = jnp.pad(table, ((0, 0), (0, EMBED_PAD - EMBED_DIM)))
    outp = _gather_sc(flat, tpad)
    return outp.reshape(BATCH, SEQ_PAD, EMBED_PAD)[:, :SEQ, :EMBED_DIM]


# speedup vs baseline: 2.4720x; 1.5449x over previous
import functools

import jax
import jax.numpy as jnp
from jax import lax
from jax.experimental import pallas as pl
from jax.experimental.pallas import tpu as pltpu
from jax.experimental.pallas import tpu_sc as plsc

VOCAB = 100000
EMBED_DIM = 300
EMBED_PAD = 384
BATCH = 4096
SEQ = 50
SEQ_PAD = 56

N_IDX = BATCH * SEQ_PAD
NUM_WORKERS = 32
PER_WORKER = N_IDX // NUM_WORKERS
CHUNK = 128
N_CHUNKS = PER_WORKER // CHUNK

_mesh = plsc.VectorSubcoreMesh(core_axis_name="c", subcore_axis_name="s")


@functools.partial(
    pl.kernel,
    mesh=_mesh,
    out_type=jax.ShapeDtypeStruct((N_IDX, EMBED_PAD), jnp.float32),
    scratch_types=[
        pltpu.VMEM((CHUNK,), jnp.int32),
        pltpu.VMEM((CHUNK, EMBED_PAD), jnp.float32),
        pltpu.SemaphoreType.DMA,
        pltpu.SemaphoreType.DMA,
        pltpu.SemaphoreType.DMA,
    ],
)
def _gather_sc3(idx_hbm, table_hbm, tail_hbm, out_hbm, idx_v, rows_v,
                s0, s1, s2):
    wid = lax.axis_index("s") * 2 + lax.axis_index("c")
    base = wid * PER_WORKER

    def chunk_body(j, carry):
        off = base + j * CHUNK
        pltpu.sync_copy(idx_hbm.at[pl.ds(off, CHUNK)], idx_v)
        c0 = pltpu.async_copy(
            table_hbm.at[:, pl.ds(0, 128)].at[idx_v],
            rows_v.at[:, pl.ds(0, 128)], s0)
        c1 = pltpu.async_copy(
            table_hbm.at[:, pl.ds(128, 128)].at[idx_v],
            rows_v.at[:, pl.ds(128, 128)], s1)
        c2 = pltpu.async_copy(
            tail_hbm.at[idx_v], rows_v.at[:, pl.ds(256, 128)], s2)
        c0.wait()
        c1.wait()
        c2.wait()
        pltpu.sync_copy(rows_v, out_hbm.at[pl.ds(off, CHUNK)])
        return carry

    lax.fori_loop(0, N_CHUNKS, chunk_body, 0)


_TAIL_ROWS = 2000


def _tail_body(t_ref, o_ref):
    o_ref[:, :EMBED_DIM - 256] = t_ref[:, 256:]
    o_ref[:, EMBED_DIM - 256:] = jnp.zeros(
        (_TAIL_ROWS, 128 - (EMBED_DIM - 256)), jnp.float32)


_tc_tail = pl.pallas_call(
    _tail_body,
    grid=(VOCAB // _TAIL_ROWS,),
    in_specs=[pl.BlockSpec((_TAIL_ROWS, EMBED_DIM), lambda i: (i, 0))],
    out_specs=pl.BlockSpec((_TAIL_ROWS, 128), lambda i: (i, 0)),
    out_shape=jax.ShapeDtypeStruct((VOCAB, 128), jnp.float32),
)


def kernel(batch, table):
    idxp = jnp.pad(batch, ((0, 0), (0, SEQ_PAD - SEQ)), mode="edge")
    flat = idxp.reshape(N_IDX)
    tail = _tc_tail(table)
    outp = _gather_sc3(flat, table, tail)
    return outp.reshape(BATCH, SEQ_PAD, EMBED_PAD)[:, :SEQ, :EMBED_DIM]


# speedup vs baseline: 2.5323x; 1.0244x over previous
import functools

import jax
import jax.numpy as jnp
from jax import lax
from jax.experimental import pallas as pl
from jax.experimental.pallas import tpu as pltpu
from jax.experimental.pallas import tpu_sc as plsc

VOCAB = 100000
EMBED_DIM = 300
EMBED_PAD = 384
BATCH = 4096
SEQ = 50
SEQ_PAD = 56

N_IDX = BATCH * SEQ_PAD
NUM_WORKERS = 32
PER_WORKER = N_IDX // NUM_WORKERS
CHUNK = 128
N_CHUNKS = PER_WORKER // CHUNK

_mesh = plsc.VectorSubcoreMesh(core_axis_name="c", subcore_axis_name="s")


@functools.partial(
    pl.kernel,
    mesh=_mesh,
    out_type=jax.ShapeDtypeStruct((N_IDX, EMBED_PAD), jnp.float32),
    scratch_types=[
        pltpu.VMEM((CHUNK,), jnp.int32),
        pltpu.VMEM((CHUNK, EMBED_PAD), jnp.float32),
        pltpu.SemaphoreType.DMA,
        pltpu.SemaphoreType.DMA,
        pltpu.SemaphoreType.DMA,
    ],
)
def _gather_sc3(idx_hbm, table_hbm, tail_hbm, out_hbm, idx_v, rows_v,
                s0, s1, s2):
    wid = lax.axis_index("s") * 2 + lax.axis_index("c")
    base = wid * PER_WORKER

    def chunk_body(j, carry):
        off = base + j * CHUNK
        pltpu.sync_copy(idx_hbm.at[pl.ds(off, CHUNK)], idx_v)
        c0 = pltpu.async_copy(
            table_hbm.at[:, pl.ds(0, 256)].at[idx_v],
            rows_v.at[:, pl.ds(0, 256)], s0)
        c2 = pltpu.async_copy(
            tail_hbm.at[idx_v], rows_v.at[:, pl.ds(256, 128)], s2)
        c0.wait()
        c2.wait()
        pltpu.sync_copy(rows_v, out_hbm.at[pl.ds(off, CHUNK)])
        return carry

    lax.fori_loop(0, N_CHUNKS, chunk_body, 0)


_TAIL_ROWS = 2000


def _tail_body(t_ref, o_ref):
    cols = lax.broadcasted_iota(jnp.int32, (_TAIL_ROWS, 128), 1)
    o_ref[...] = jnp.where(cols < EMBED_DIM - 256, t_ref[...], 0.0)


_tc_tail = pl.pallas_call(
    _tail_body,
    grid=(VOCAB // _TAIL_ROWS,),
    in_specs=[pl.BlockSpec((_TAIL_ROWS, 128), lambda i: (i, 2))],
    out_specs=pl.BlockSpec((_TAIL_ROWS, 128), lambda i: (i, 0)),
    out_shape=jax.ShapeDtypeStruct((VOCAB, 128), jnp.float32),
)


def kernel(batch, table):
    idxp = jnp.pad(batch, ((0, 0), (0, SEQ_PAD - SEQ)), mode="edge")
    flat = idxp.reshape(N_IDX)
    tail = _tc_tail(table)
    outp = _gather_sc3(flat, table, tail)
    return outp.reshape(BATCH, SEQ_PAD, EMBED_PAD)[:, :SEQ, :EMBED_DIM]
